# Initial kernel scaffold; baseline (speedup 1.0000x reference)
import jax
import jax.numpy as jnp
from jax.experimental import pallas as pl


def kernel(rois, cls_pred, reg_pred, batch_size):
    raise NotImplementedError("write your pallas kernel here")


# speedup vs baseline: 43.5655x; 43.5655x over previous
import jax
import jax.numpy as jnp
from jax.experimental import pallas as pl

_THR = 0.3
_IOU_THR = 0.5
_K = 20
_EPS = 1e-12


def _decoder_kernel(rois_ref, cls_ref, reg_ref,
                    s_ref, x1_ref, y1_ref, x2_ref, y2_ref, keep_ref):
    B, C, HW = cls_ref.shape
    f32 = jnp.float32

    rois = rois_ref[...]
    x1r = rois[:, 0:1][:, :, None]
    y1r = rois[:, 1:2][:, :, None]
    x2r = rois[:, 2:3][:, :, None]
    y2r = rois[:, 3:4][:, :, None]
    cx = (x1r + x2r) * 0.5
    cy = (y1r + y2r) * 0.5
    w = x2r - x1r
    h = y2r - y1r
    nw = w * 1.0
    nh = h * 1.0
    rx1 = cx - nw * 0.5
    ry1 = cy - nh * 0.5
    rx2 = cx + nw * 0.5
    ry2 = cy + nh * 0.5
    rw = rx2 - rx1
    rh = ry2 - ry1
    bsw = rw / 16.0
    bsh = rh / 16.0

    pos = jax.lax.broadcasted_iota(jnp.int32, (B, 1, HW), 2)
    ind_w = (pos & 15).astype(f32)
    ind_h = (pos >> 4).astype(f32)

    reg = reg_ref[...]
    ox1 = reg[:, 0 * HW:1 * HW][:, None, :]
    oy1 = reg[:, 1 * HW:2 * HW][:, None, :]
    ox2 = reg[:, 2 * HW:3 * HW][:, None, :]
    oy2 = reg[:, 3 * HW:4 * HW][:, None, :]

    bx1 = bsw * (ox1 + ind_w + 0.5) + rx1
    by1 = bsh * (oy1 + ind_h + 0.5) + ry1
    bx2 = bsw * (ox2 + ind_w + 0.5) + rx1
    by2 = bsh * (oy2 + ind_h + 0.5) + ry1

    cls = cls_ref[...]
    zero3 = jnp.zeros((B, C, HW), f32)
    cur = jnp.where(cls > _THR, cls, zero3)

    iota3 = jax.lax.broadcasted_iota(jnp.int32, (B, C, HW), 2)
    svals = []
    g1l, g2l, g3l, g4l = [], [], [], []
    for _ in range(_K):
        v = jnp.max(cur, axis=2, keepdims=True)
        eq = cur == v
        idx = jnp.min(jnp.where(eq, iota3, HW), axis=2, keepdims=True)
        oh = iota3 == idx
        cur = jnp.where(oh, -1.0, cur)
        ohf = jnp.where(oh, 1.0, 0.0)
        tk = v > _THR
        g1 = jnp.sum(ohf * bx1, axis=2, keepdims=True)
        g2 = jnp.sum(ohf * by1, axis=2, keepdims=True)
        g3 = jnp.sum(ohf * bx2, axis=2, keepdims=True)
        g4 = jnp.sum(ohf * by2, axis=2, keepdims=True)
        zc = jnp.zeros_like(g1)
        svals.append(v)
        g1l.append(jnp.where(tk, g1, zc))
        g2l.append(jnp.where(tk, g2, zc))
        g3l.append(jnp.where(tk, g3, zc))
        g4l.append(jnp.where(tk, g4, zc))

    s = jnp.concatenate(svals, axis=2)
    bx = jnp.concatenate(g1l, axis=2)
    by = jnp.concatenate(g2l, axis=2)
    bX = jnp.concatenate(g3l, axis=2)
    bY = jnp.concatenate(g4l, axis=2)

    def to_row(a):
        return jnp.concatenate([a[:, c:c + 1, :] for c in range(C)], axis=2)

    s2 = to_row(s)
    bx_2 = to_row(bx)
    by_2 = to_row(by)
    bX_2 = to_row(bX)
    bY_2 = to_row(bY)
    mc = jnp.maximum(jnp.maximum(bx_2, by_2), jnp.maximum(bX_2, bY_2))
    mc = jnp.max(mc, axis=2, keepdims=True) + 1.0
    CK = C * _K
    ci = jax.lax.broadcasted_iota(jnp.int32, (B, 1, CK), 2)
    labf = (ci // _K).astype(f32)
    cif = ci.astype(f32)
    offv = labf * mc
    obx1 = bx_2 + offv
    oby1 = by_2 + offv
    obx2 = bX_2 + offv
    oby2 = bY_2 + offv
    area = (obx2 - obx1) * (oby2 - oby1)
    validf = jnp.where(s2 >= _THR, 1.0, 0.0)

    supp = jnp.zeros((B, 1, CK), f32)
    keep = jnp.zeros((B, 1, CK), f32)
    for _it in range(_K):
        candf = validf * (1.0 - supp) * (1.0 - keep)
        ms = jnp.where(candf > 0.5, s2, -1.0)
        v = jnp.max(ms, axis=2, keepdims=True)
        takef = jnp.where(v >= _THR, 1.0, 0.0)
        eq = ms == v
        sidx = jnp.min(jnp.where(eq, cif, 1e9), axis=2, keepdims=True)
        oh = cif == sidx
        ohf = jnp.where(oh, 1.0, 0.0)
        sx1 = jnp.sum(ohf * obx1, axis=2, keepdims=True)
        sy1 = jnp.sum(ohf * oby1, axis=2, keepdims=True)
        sx2 = jnp.sum(ohf * obx2, axis=2, keepdims=True)
        sy2 = jnp.sum(ohf * oby2, axis=2, keepdims=True)
        ix1 = jnp.maximum(obx1, sx1)
        iy1 = jnp.maximum(oby1, sy1)
        ix2 = jnp.minimum(obx2, sx2)
        iy2 = jnp.minimum(oby2, sy2)
        iw = jnp.maximum(ix2 - ix1, 0.0)
        ih = jnp.maximum(iy2 - iy1, 0.0)
        inter = iw * ih
        sarea = (sx2 - sx1) * (sy2 - sy1)
        union = area + sarea - inter
        iou = inter / jnp.maximum(union, _EPS)
        keep = jnp.maximum(keep, ohf * takef)
        newsupp = jnp.where(iou > _IOU_THR, 1.0, 0.0)
        supp = jnp.maximum(supp, newsupp * takef)

    s_ref[...] = s
    x1_ref[...] = bx
    y1_ref[...] = by
    x2_ref[...] = bX
    y2_ref[...] = bY
    keep_ref[...] = keep


def kernel(rois, cls_pred, reg_pred, batch_size):
    R, C, H, W = cls_pred.shape
    HW = H * W
    cls3 = cls_pred.reshape(R, C, HW)
    reg2 = reg_pred.reshape(R, 4 * HW)
    B = 128
    grid = (R // B,)
    o3 = pl.BlockSpec((B, C, _K), lambda i: (i, 0, 0))
    ok = pl.BlockSpec((B, 1, C * _K), lambda i: (i, 0, 0))
    outs = pl.pallas_call(
        _decoder_kernel,
        grid=grid,
        in_specs=[
            pl.BlockSpec((B, 4), lambda i: (i, 0)),
            pl.BlockSpec((B, C, HW), lambda i: (i, 0, 0)),
            pl.BlockSpec((B, 4 * HW), lambda i: (i, 0)),
        ],
        out_specs=[o3, o3, o3, o3, o3, ok],
        out_shape=[
            jax.ShapeDtypeStruct((R, C, _K), jnp.float32),
            jax.ShapeDtypeStruct((R, C, _K), jnp.float32),
            jax.ShapeDtypeStruct((R, C, _K), jnp.float32),
            jax.ShapeDtypeStruct((R, C, _K), jnp.float32),
            jax.ShapeDtypeStruct((R, C, _K), jnp.float32),
            jax.ShapeDtypeStruct((R, 1, C * _K), jnp.float32),
        ],
    )(rois, cls3, reg2)
    s3, x1, y1, x2, y2, kp = outs
    res_scores = s3.reshape(R, C * _K)
    res_boxes = jnp.stack([x1, y1, x2, y2], axis=-1).reshape(R, C * _K, 4)
    res_labels = jnp.broadcast_to(
        jnp.broadcast_to(jnp.arange(C, dtype=jnp.int32)[:, None], (C, _K)).reshape(1, C * _K),
        (R, C * _K))
    keep_all = kp.reshape(R, C * _K) != 0
    return res_boxes, res_scores, res_labels, keep_all
